# baseline (speedup 1.0000x reference)
import functools

import jax
import jax.numpy as jnp
from jax import lax
from jax.experimental import pallas as pl
from jax.experimental.pallas import tpu as pltpu
from jax.experimental.pallas import tpu_sc as plsc

N = 10000
G = 64
D_IN = 128
D_H = 512
NC, NS, LANES = 2, 16, 16
NW = NC * NS
CHUNK = 128
NBUF = 2
DCH = 128
NP = 10240
PR = 10240
RPT = PR // NS
E_RAW = 320000
E_PAD = -(-E_RAW // (NW * NBUF * CHUNK * 2)) * (NW * NBUF * CHUNK * 2)
EPW = E_PAD // NW
EPT = E_PAD // NS

_MESH = dict(core_axis_name="c", subcore_axis_name="s")


def _edge_pass(src_hbm, dst_hbm, tbl_hbm, acc, sidx, didx, rows,
               isems, gsems, ssems, base, T):
    def idx_load(it, j):
        off = jnp.minimum(base + it * CHUNK, E_PAD - CHUNK)
        pltpu.async_copy(src_hbm.at[pl.ds(off, CHUNK)], sidx.at[j], isems[j])
        pltpu.async_copy(dst_hbm.at[pl.ds(off, CHUNK)], didx.at[j], isems[j])

    def idx_wait(j):
        pltpu.make_async_copy(src_hbm.at[pl.ds(0, CHUNK)], sidx.at[j],
                              isems[j]).wait()
        pltpu.make_async_copy(dst_hbm.at[pl.ds(0, CHUNK)], didx.at[j],
                              isems[j]).wait()

    def group(i0, reload):
        gds = []
        for j in range(NBUF):
            idx_wait(j)
            gds.append(pltpu.async_copy(tbl_hbm.at[sidx.at[j]], rows.at[j],
                                        gsems[j]))
        sds = []
        for j in range(NBUF):
            gds[j].wait()
            sds.append(pltpu.async_copy(rows.at[j], acc.at[didx.at[j]],
                                        ssems[j], add=True))
        for j in range(NBUF):
            sds[j].wait()
        if reload:
            for j in range(NBUF):
                idx_load(i0 + NBUF + j, j)

    for j in range(NBUF):
        idx_load(j, j)

    def body(g, carry):
        group(NBUF * g, True)
        return carry

    lax.fori_loop(0, T // NBUF - 1, body, 0)
    group(T - NBUF, False)


def _deg_body(dst_hbm, zn_hbm, d0_hbm, d1_hbm, didx, ones_v, acc, *sems):
    c = lax.axis_index("c")
    s = lax.axis_index("s")
    wid = c * NS + s
    isems, ssems = sems[:4], sems[4:]
    pltpu.sync_copy(zn_hbm.at[pl.ds(s * (NP // NS), NP // NS)],
                    acc.at[pl.ds(s * (NP // NS), NP // NS)])
    for i in range(DCH // LANES):
        ones_v[pl.ds(i * LANES, LANES)] = jnp.full((LANES,), 1.0, jnp.float32)
    plsc.subcore_barrier()

    T = EPW // DCH
    base = wid * EPW

    def idx_load(it, j):
        off = jnp.minimum(base + it * DCH, E_PAD - DCH)
        pltpu.async_copy(dst_hbm.at[pl.ds(off, DCH)], didx.at[j], isems[j])

    def group(i0, reload):
        sds = []
        for j in range(4):
            pltpu.make_async_copy(dst_hbm.at[pl.ds(0, DCH)], didx.at[j],
                                  isems[j]).wait()
            sds.append(pltpu.async_copy(ones_v, acc.at[didx.at[j]],
                                        ssems[j], add=True))
        for j in range(4):
            sds[j].wait()
        if reload:
            for j in range(4):
                idx_load(i0 + 4 + j, j)

    for j in range(4):
        idx_load(j, j)

    def body(g, carry):
        group(4 * g, True)
        return carry

    lax.fori_loop(0, T // 4 - 1, body, 0)
    group(T - 4, False)
    plsc.subcore_barrier()

    @pl.when(jnp.logical_and(s == 0, c == 0))
    def _():
        pltpu.sync_copy(acc, d0_hbm)

    @pl.when(jnp.logical_and(s == 0, c == 1))
    def _():
        pltpu.sync_copy(acc, d1_hbm)


def _make_deg():
    return pl.kernel(
        _deg_body,
        out_type=(jax.ShapeDtypeStruct((NP,), jnp.float32),
                  jax.ShapeDtypeStruct((NP,), jnp.float32)),
        mesh=plsc.VectorSubcoreMesh(**_MESH),
        scratch_types=[
            pltpu.VMEM((4, DCH), jnp.int32),
            pltpu.VMEM((DCH,), jnp.float32),
            pltpu.VMEM_SHARED((NP,), jnp.float32),
        ] + [pltpu.SemaphoreType.DMA] * 8,
    )


def _mp1_body(src_hbm, dst_hbm, xt_hbm, z_hbm, p0_hbm, p1_hbm,
              sidx, didx, rows, acc, *sems):
    c = lax.axis_index("c")
    s = lax.axis_index("s")
    wid = c * NS + s
    r0 = s * RPT

    @pl.when(c == 0)
    def _():
        pltpu.sync_copy(xt_hbm.at[pl.ds(r0, RPT)], acc.at[pl.ds(r0, RPT)])

    @pl.when(c == 1)
    def _():
        pltpu.sync_copy(z_hbm.at[pl.ds(r0, RPT)], acc.at[pl.ds(r0, RPT)])

    plsc.subcore_barrier()
    _edge_pass(src_hbm, dst_hbm, xt_hbm, acc, sidx, didx, rows,
               sems[:NBUF], sems[NBUF:2 * NBUF], sems[2 * NBUF:],
               wid * EPW, EPW // CHUNK)
    plsc.subcore_barrier()

    @pl.when(c == 0)
    def _():
        pltpu.sync_copy(acc.at[pl.ds(r0, RPT)], p0_hbm.at[pl.ds(r0, RPT)])

    @pl.when(c == 1)
    def _():
        pltpu.sync_copy(acc.at[pl.ds(r0, RPT)], p1_hbm.at[pl.ds(r0, RPT)])


def _make_mp1():
    return pl.kernel(
        _mp1_body,
        out_type=(jax.ShapeDtypeStruct((PR, D_IN), jnp.float32),
                  jax.ShapeDtypeStruct((PR, D_IN), jnp.float32)),
        mesh=plsc.VectorSubcoreMesh(**_MESH),
        scratch_types=[
            pltpu.VMEM((NBUF, CHUNK), jnp.int32),
            pltpu.VMEM((NBUF, CHUNK), jnp.int32),
            pltpu.VMEM((NBUF, CHUNK, D_IN), jnp.float32),
            pltpu.VMEM_SHARED((PR, D_IN), jnp.float32),
        ] + [pltpu.SemaphoreType.DMA] * (3 * NBUF),
    )


def _mp2_body(src_hbm, dst_hbm, m0, m1, m2, m3, v0, v1, v2, v3,
              sidx, didx, rows, acc, *sems):
    c = lax.axis_index("c")
    s = lax.axis_index("s")
    r0 = s * RPT

    def quarter(m_hbm, v_hbm):
        pltpu.sync_copy(m_hbm.at[pl.ds(r0, RPT)], acc.at[pl.ds(r0, RPT)])
        plsc.subcore_barrier()
        _edge_pass(src_hbm, dst_hbm, m_hbm, acc, sidx, didx, rows,
                   sems[:NBUF], sems[NBUF:2 * NBUF], sems[2 * NBUF:],
                   s * EPT, EPT // CHUNK)
        plsc.subcore_barrier()
        pltpu.sync_copy(acc.at[pl.ds(r0, RPT)], v_hbm.at[pl.ds(r0, RPT)])

    @pl.when(c == 0)
    def _():
        quarter(m0, v0)
        quarter(m1, v1)

    @pl.when(c == 1)
    def _():
        quarter(m2, v2)
        quarter(m3, v3)


def _make_mp2():
    vec = jax.ShapeDtypeStruct((PR, D_IN), jnp.float32)
    return pl.kernel(
        _mp2_body,
        out_type=(vec, vec, vec, vec),
        mesh=plsc.VectorSubcoreMesh(**_MESH),
        scratch_types=[
            pltpu.VMEM((NBUF, CHUNK), jnp.int32),
            pltpu.VMEM((NBUF, CHUNK), jnp.int32),
            pltpu.VMEM((NBUF, CHUNK, D_IN), jnp.float32),
            pltpu.VMEM_SHARED((PR, D_IN), jnp.float32),
        ] + [pltpu.SemaphoreType.DMA] * (3 * NBUF),
    )


def _prep_body(d0_ref, d1_ref, x_ref, r_ref, xt_ref):
    d = d0_ref[...] + d1_ref[...] + 1.0
    rr = lax.rsqrt(d)
    r_ref[...] = rr
    xt_ref[0:N, :] = x_ref[...] * rr[:N]
    xt_ref[N:PR, :] = jnp.zeros((PR - N, D_IN), jnp.float32)


def _make_prep():
    return pl.pallas_call(
        _prep_body,
        out_shape=(jax.ShapeDtypeStruct((PR, 1), jnp.float32),
                   jax.ShapeDtypeStruct((PR, D_IN), jnp.float32)),
    )


BC = 2048


def _mm_body(p0_ref, p1_ref, r_ref, W1_ref, b1_ref, W2_ref,
             m0_ref, m1_ref, m2_ref, m3_ref):
    r = r_ref[...]
    u = (p0_ref[...] + p1_ref[...]) * r
    h = jnp.dot(u, W1_ref[...], preferred_element_type=jnp.float32)
    h = jnp.maximum(h + b1_ref[...][None, :], 0.0)
    mt = jnp.dot(h, W2_ref[...], preferred_element_type=jnp.float32) * r
    m0_ref[...] = mt[:, 0:128]
    m1_ref[...] = mt[:, 128:256]
    m2_ref[...] = mt[:, 256:384]
    m3_ref[...] = mt[:, 384:512]


def _make_mm():
    blk = pl.BlockSpec((BC, D_IN), lambda i: (i, 0))
    return pl.pallas_call(
        _mm_body,
        grid=(PR // BC,),
        in_specs=[
            blk, blk,
            pl.BlockSpec((BC, 1), lambda i: (i, 0)),
            pl.BlockSpec((D_IN, D_H), lambda i: (0, 0)),
            pl.BlockSpec((D_H,), lambda i: (0,)),
            pl.BlockSpec((D_H, D_H), lambda i: (0, 0)),
        ],
        out_specs=[blk, blk, blk, blk],
        out_shape=tuple(jax.ShapeDtypeStruct((PR, D_IN), jnp.float32)
                        for _ in range(4)),
    )


def _pool_body(v0_ref, v1_ref, v2_ref, v3_ref, r_ref, b2_ref, bt_ref,
               out_ref, sums, counts):
    i = pl.program_id(0)

    @pl.when(i == 0)
    def _():
        sums[...] = jnp.zeros_like(sums)
        counts[...] = jnp.zeros_like(counts)

    bb = bt_ref[...]
    gi = lax.broadcasted_iota(jnp.int32, (BC, G), 1)
    oh = (bb == gi).astype(jnp.float32)
    ones = jnp.ones((BC, 1), jnp.float32)
    dn = (((0,), (0,)), ((), ()))
    counts[...] += lax.dot_general(oh, ones, dn,
                                   preferred_element_type=jnp.float32)
    r = r_ref[...]
    for q, v_ref in enumerate((v0_ref, v1_ref, v2_ref, v3_ref)):
        h2 = jnp.maximum(v_ref[...] * r + b2_ref[...][None,
                                                      q * 128:(q + 1) * 128],
                         0.0)
        sums[q] += lax.dot_general(oh, h2, dn,
                                   preferred_element_type=jnp.float32)

    @pl.when(i == pl.num_programs(0) - 1)
    def _():
        cts = jnp.maximum(counts[...], 1.0)
        for q in range(4):
            out_ref[:, q * 128:(q + 1) * 128] = sums[q] / cts


def _make_pool():
    blk = pl.BlockSpec((BC, D_IN), lambda i: (i, 0))
    return pl.pallas_call(
        _pool_body,
        grid=(PR // BC,),
        in_specs=[
            blk, blk, blk, blk,
            pl.BlockSpec((BC, 1), lambda i: (i, 0)),
            pl.BlockSpec((D_H,), lambda i: (0,)),
            pl.BlockSpec((BC, 1), lambda i: (i, 0)),
        ],
        out_specs=pl.BlockSpec((G, D_H), lambda i: (0, 0)),
        out_shape=jax.ShapeDtypeStruct((G, D_H), jnp.float32),
        scratch_shapes=[
            pltpu.VMEM((4, G, D_IN), jnp.float32),
            pltpu.VMEM((G, 1), jnp.float32),
        ],
    )


@functools.partial(jax.jit, static_argnums=())
def _run(x, src, dst, batch2d, W1, b1, W2, b2):
    pad = E_PAD - src.shape[0]
    ar = lax.iota(jnp.int32, pad)
    src_p = jnp.concatenate([src, ar % 64])
    dst_p = jnp.concatenate([dst, N + (ar % 64)])
    zn = jnp.zeros((NP,), jnp.float32)
    z128 = jnp.zeros((PR, D_IN), jnp.float32)

    d0, d1 = _make_deg()(dst_p, zn)
    r, xt = _make_prep()(d0.reshape(NP, 1), d1.reshape(NP, 1), x)
    p0, p1 = _make_mp1()(src_p, dst_p, xt, z128)
    m0, m1, m2, m3 = _make_mm()(p0, p1, r, W1, b1, W2)
    v0, v1, v2, v3 = _make_mp2()(src_p, dst_p, m0, m1, m2, m3)
    return _make_pool()(v0, v1, v2, v3, r, b2, batch2d)


def kernel(x, edge_index, batch, W1, b1, W2, b2):
    src = edge_index[0].astype(jnp.int32)
    dst = edge_index[1].astype(jnp.int32)
    bpad = jnp.full((PR - N,), G, jnp.int32)
    batch2d = jnp.concatenate([batch.astype(jnp.int32), bpad]).reshape(PR, 1)
    return _run(x, src, dst, batch2d, W1, b1, W2, b2)


# speedup vs baseline: 1.0576x; 1.0576x over previous
import functools

import jax
import jax.numpy as jnp
from jax import lax
from jax.experimental import pallas as pl
from jax.experimental.pallas import tpu as pltpu
from jax.experimental.pallas import tpu_sc as plsc

N = 10000
G = 64
D_IN = 128
D_H = 512
NC, NS, LANES = 2, 16, 16
NW = NC * NS
CHUNK = 112
NBUF = 3
DCH = CHUNK
NP = 10240
PR = 10240
RPT = PR // NS
E_RAW = 320000
E_PAD = -(-E_RAW // (NW * NBUF * CHUNK * 2)) * (NW * NBUF * CHUNK * 2)
EPW = E_PAD // NW
EPT = E_PAD // NS

_MESH = dict(core_axis_name="c", subcore_axis_name="s")


def _edge_pass(src_hbm, dst_hbm, tbl_hbm, acc, sidx, didx, rows,
               isems, gsems, ssems, base, T):
    def idx_load(it, j):
        off = jnp.minimum(base + it * CHUNK, E_PAD - CHUNK)
        pltpu.async_copy(src_hbm.at[pl.ds(off, CHUNK)], sidx.at[j], isems[j])
        pltpu.async_copy(dst_hbm.at[pl.ds(off, CHUNK)], didx.at[j], isems[j])

    def idx_wait(j):
        pltpu.make_async_copy(src_hbm.at[pl.ds(0, CHUNK)], sidx.at[j],
                              isems[j]).wait()
        pltpu.make_async_copy(dst_hbm.at[pl.ds(0, CHUNK)], didx.at[j],
                              isems[j]).wait()

    def group(i0, reload):
        gds = []
        for j in range(NBUF):
            idx_wait(j)
            gds.append(pltpu.async_copy(tbl_hbm.at[sidx.at[j]], rows.at[j],
                                        gsems[j]))
        sds = []
        for j in range(NBUF):
            gds[j].wait()
            sds.append(pltpu.async_copy(rows.at[j], acc.at[didx.at[j]],
                                        ssems[j], add=True))
        for j in range(NBUF):
            sds[j].wait()
        if reload:
            for j in range(NBUF):
                idx_load(i0 + NBUF + j, j)

    for j in range(NBUF):
        idx_load(j, j)

    def body(g, carry):
        group(NBUF * g, True)
        return carry

    lax.fori_loop(0, T // NBUF - 1, body, 0)
    group(T - NBUF, False)


def _deg_body(dst_hbm, zn_hbm, d0_hbm, d1_hbm, didx, ones_v, acc, *sems):
    c = lax.axis_index("c")
    s = lax.axis_index("s")
    wid = c * NS + s
    isems, ssems = sems[:NBUF], sems[NBUF:]
    pltpu.sync_copy(zn_hbm.at[pl.ds(s * (NP // NS), NP // NS)],
                    acc.at[pl.ds(s * (NP // NS), NP // NS)])
    for i in range(DCH // LANES):
        ones_v[pl.ds(i * LANES, LANES)] = jnp.full((LANES,), 1.0, jnp.float32)
    plsc.subcore_barrier()

    T = EPW // DCH
    base = wid * EPW

    def idx_load(it, j):
        off = jnp.minimum(base + it * DCH, E_PAD - DCH)
        pltpu.async_copy(dst_hbm.at[pl.ds(off, DCH)], didx.at[j], isems[j])

    def group(i0, reload):
        sds = []
        for j in range(NBUF):
            pltpu.make_async_copy(dst_hbm.at[pl.ds(0, DCH)], didx.at[j],
                                  isems[j]).wait()
            sds.append(pltpu.async_copy(ones_v, acc.at[didx.at[j]],
                                        ssems[j], add=True))
        for j in range(NBUF):
            sds[j].wait()
        if reload:
            for j in range(NBUF):
                idx_load(i0 + NBUF + j, j)

    for j in range(NBUF):
        idx_load(j, j)

    def body(g, carry):
        group(NBUF * g, True)
        return carry

    lax.fori_loop(0, T // NBUF - 1, body, 0)
    group(T - NBUF, False)
    plsc.subcore_barrier()

    @pl.when(jnp.logical_and(s == 0, c == 0))
    def _():
        pltpu.sync_copy(acc, d0_hbm)

    @pl.when(jnp.logical_and(s == 0, c == 1))
    def _():
        pltpu.sync_copy(acc, d1_hbm)


def _make_deg():
    return pl.kernel(
        _deg_body,
        out_type=(jax.ShapeDtypeStruct((NP,), jnp.float32),
                  jax.ShapeDtypeStruct((NP,), jnp.float32)),
        mesh=plsc.VectorSubcoreMesh(**_MESH),
        scratch_types=[
            pltpu.VMEM((NBUF, DCH), jnp.int32),
            pltpu.VMEM((DCH,), jnp.float32),
            pltpu.VMEM_SHARED((NP,), jnp.float32),
        ] + [pltpu.SemaphoreType.DMA] * (2 * NBUF),
    )


def _mp1_body(src_hbm, dst_hbm, xt_hbm, z_hbm, p0_hbm, p1_hbm,
              sidx, didx, rows, acc, *sems):
    c = lax.axis_index("c")
    s = lax.axis_index("s")
    wid = c * NS + s
    r0 = s * RPT

    @pl.when(c == 0)
    def _():
        pltpu.sync_copy(xt_hbm.at[pl.ds(r0, RPT)], acc.at[pl.ds(r0, RPT)])

    @pl.when(c == 1)
    def _():
        pltpu.sync_copy(z_hbm.at[pl.ds(r0, RPT)], acc.at[pl.ds(r0, RPT)])

    plsc.subcore_barrier()
    _edge_pass(src_hbm, dst_hbm, xt_hbm, acc, sidx, didx, rows,
               sems[:NBUF], sems[NBUF:2 * NBUF], sems[2 * NBUF:],
               wid * EPW, EPW // CHUNK)
    plsc.subcore_barrier()

    @pl.when(c == 0)
    def _():
        pltpu.sync_copy(acc.at[pl.ds(r0, RPT)], p0_hbm.at[pl.ds(r0, RPT)])

    @pl.when(c == 1)
    def _():
        pltpu.sync_copy(acc.at[pl.ds(r0, RPT)], p1_hbm.at[pl.ds(r0, RPT)])


def _make_mp1():
    return pl.kernel(
        _mp1_body,
        out_type=(jax.ShapeDtypeStruct((PR, D_IN), jnp.float32),
                  jax.ShapeDtypeStruct((PR, D_IN), jnp.float32)),
        mesh=plsc.VectorSubcoreMesh(**_MESH),
        scratch_types=[
            pltpu.VMEM((NBUF, CHUNK), jnp.int32),
            pltpu.VMEM((NBUF, CHUNK), jnp.int32),
            pltpu.VMEM((NBUF, CHUNK, D_IN), jnp.float32),
            pltpu.VMEM_SHARED((PR, D_IN), jnp.float32),
        ] + [pltpu.SemaphoreType.DMA] * (3 * NBUF),
    )


def _mp2_body(src_hbm, dst_hbm, m0, m1, m2, m3, v0, v1, v2, v3,
              sidx, didx, rows, acc, *sems):
    c = lax.axis_index("c")
    s = lax.axis_index("s")
    r0 = s * RPT

    def quarter(m_hbm, v_hbm):
        pltpu.sync_copy(m_hbm.at[pl.ds(r0, RPT)], acc.at[pl.ds(r0, RPT)])
        plsc.subcore_barrier()
        _edge_pass(src_hbm, dst_hbm, m_hbm, acc, sidx, didx, rows,
                   sems[:NBUF], sems[NBUF:2 * NBUF], sems[2 * NBUF:],
                   s * EPT, EPT // CHUNK)
        plsc.subcore_barrier()
        pltpu.sync_copy(acc.at[pl.ds(r0, RPT)], v_hbm.at[pl.ds(r0, RPT)])

    @pl.when(c == 0)
    def _():
        quarter(m0, v0)
        quarter(m1, v1)

    @pl.when(c == 1)
    def _():
        quarter(m2, v2)
        quarter(m3, v3)


def _make_mp2():
    vec = jax.ShapeDtypeStruct((PR, D_IN), jnp.float32)
    return pl.kernel(
        _mp2_body,
        out_type=(vec, vec, vec, vec),
        mesh=plsc.VectorSubcoreMesh(**_MESH),
        scratch_types=[
            pltpu.VMEM((NBUF, CHUNK), jnp.int32),
            pltpu.VMEM((NBUF, CHUNK), jnp.int32),
            pltpu.VMEM((NBUF, CHUNK, D_IN), jnp.float32),
            pltpu.VMEM_SHARED((PR, D_IN), jnp.float32),
        ] + [pltpu.SemaphoreType.DMA] * (3 * NBUF),
    )


def _prep_body(d0_ref, d1_ref, x_ref, r_ref, xt_ref):
    d = d0_ref[...] + d1_ref[...] + 1.0
    rr = lax.rsqrt(d)
    r_ref[...] = rr
    xt_ref[0:N, :] = x_ref[...] * rr[:N]
    xt_ref[N:PR, :] = jnp.zeros((PR - N, D_IN), jnp.float32)


def _make_prep():
    return pl.pallas_call(
        _prep_body,
        out_shape=(jax.ShapeDtypeStruct((PR, 1), jnp.float32),
                   jax.ShapeDtypeStruct((PR, D_IN), jnp.float32)),
    )


BC = 2048


def _mm_body(p0_ref, p1_ref, r_ref, W1_ref, b1_ref, W2_ref,
             m0_ref, m1_ref, m2_ref, m3_ref):
    r = r_ref[...]
    u = (p0_ref[...] + p1_ref[...]) * r
    h = jnp.dot(u, W1_ref[...], preferred_element_type=jnp.float32)
    h = jnp.maximum(h + b1_ref[...][None, :], 0.0)
    mt = jnp.dot(h, W2_ref[...], preferred_element_type=jnp.float32) * r
    m0_ref[...] = mt[:, 0:128]
    m1_ref[...] = mt[:, 128:256]
    m2_ref[...] = mt[:, 256:384]
    m3_ref[...] = mt[:, 384:512]


def _make_mm():
    blk = pl.BlockSpec((BC, D_IN), lambda i: (i, 0))
    return pl.pallas_call(
        _mm_body,
        grid=(PR // BC,),
        in_specs=[
            blk, blk,
            pl.BlockSpec((BC, 1), lambda i: (i, 0)),
            pl.BlockSpec((D_IN, D_H), lambda i: (0, 0)),
            pl.BlockSpec((D_H,), lambda i: (0,)),
            pl.BlockSpec((D_H, D_H), lambda i: (0, 0)),
        ],
        out_specs=[blk, blk, blk, blk],
        out_shape=tuple(jax.ShapeDtypeStruct((PR, D_IN), jnp.float32)
                        for _ in range(4)),
    )


def _pool_body(v0_ref, v1_ref, v2_ref, v3_ref, r_ref, b2_ref, bt_ref,
               out_ref, sums, counts):
    i = pl.program_id(0)

    @pl.when(i == 0)
    def _():
        sums[...] = jnp.zeros_like(sums)
        counts[...] = jnp.zeros_like(counts)

    bb = bt_ref[...]
    gi = lax.broadcasted_iota(jnp.int32, (BC, G), 1)
    oh = (bb == gi).astype(jnp.float32)
    ones = jnp.ones((BC, 1), jnp.float32)
    dn = (((0,), (0,)), ((), ()))
    counts[...] += lax.dot_general(oh, ones, dn,
                                   preferred_element_type=jnp.float32)
    r = r_ref[...]
    for q, v_ref in enumerate((v0_ref, v1_ref, v2_ref, v3_ref)):
        h2 = jnp.maximum(v_ref[...] * r + b2_ref[...][None,
                                                      q * 128:(q + 1) * 128],
                         0.0)
        sums[q] += lax.dot_general(oh, h2, dn,
                                   preferred_element_type=jnp.float32)

    @pl.when(i == pl.num_programs(0) - 1)
    def _():
        cts = jnp.maximum(counts[...], 1.0)
        for q in range(4):
            out_ref[:, q * 128:(q + 1) * 128] = sums[q] / cts


def _make_pool():
    blk = pl.BlockSpec((BC, D_IN), lambda i: (i, 0))
    return pl.pallas_call(
        _pool_body,
        grid=(PR // BC,),
        in_specs=[
            blk, blk, blk, blk,
            pl.BlockSpec((BC, 1), lambda i: (i, 0)),
            pl.BlockSpec((D_H,), lambda i: (0,)),
            pl.BlockSpec((BC, 1), lambda i: (i, 0)),
        ],
        out_specs=pl.BlockSpec((G, D_H), lambda i: (0, 0)),
        out_shape=jax.ShapeDtypeStruct((G, D_H), jnp.float32),
        scratch_shapes=[
            pltpu.VMEM((4, G, D_IN), jnp.float32),
            pltpu.VMEM((G, 1), jnp.float32),
        ],
    )


@functools.partial(jax.jit, static_argnums=())
def _run(x, src, dst, batch2d, W1, b1, W2, b2):
    pad = E_PAD - src.shape[0]
    ar = lax.iota(jnp.int32, pad)
    src_p = jnp.concatenate([src, ar % 64])
    dst_p = jnp.concatenate([dst, N + (ar % 64)])
    zn = jnp.zeros((NP,), jnp.float32)
    z128 = jnp.zeros((PR, D_IN), jnp.float32)

    d0, d1 = _make_deg()(dst_p, zn)
    r, xt = _make_prep()(d0.reshape(NP, 1), d1.reshape(NP, 1), x)
    p0, p1 = _make_mp1()(src_p, dst_p, xt, z128)
    m0, m1, m2, m3 = _make_mm()(p0, p1, r, W1, b1, W2)
    v0, v1, v2, v3 = _make_mp2()(src_p, dst_p, m0, m1, m2, m3)
    return _make_pool()(v0, v1, v2, v3, r, b2, batch2d)


def kernel(x, edge_index, batch, W1, b1, W2, b2):
    src = edge_index[0].astype(jnp.int32)
    dst = edge_index[1].astype(jnp.int32)
    bpad = jnp.full((PR - N,), G, jnp.int32)
    batch2d = jnp.concatenate([batch.astype(jnp.int32), bpad]).reshape(PR, 1)
    return _run(x, src, dst, batch2d, W1, b1, W2, b2)


# speedup vs baseline: 1.1768x; 1.1126x over previous
import functools

import jax
import jax.numpy as jnp
from jax import lax
from jax.experimental import pallas as pl
from jax.experimental.pallas import tpu as pltpu
from jax.experimental.pallas import tpu_sc as plsc

N = 10000
G = 64
D_IN = 128
D_H = 512
NC, NS, LANES = 2, 16, 16
NW = NC * NS
CHUNK = 112
NBUF = 3
DCH = CHUNK
NP = 10240
PR = 10240
RPT = PR // NS
E_RAW = 320000
E_PAD = -(-E_RAW // (NW * NBUF * CHUNK * 2)) * (NW * NBUF * CHUNK * 2)
EPW = E_PAD // NW
EPT = E_PAD // NS

_MESH = dict(core_axis_name="c", subcore_axis_name="s")


def _edge_pass(src_hbm, dst_hbm, tbl_hbm, acc, sidx, didx, rows,
               isems, gsems, ssems, base, T):
    def idx_load(it, j):
        off = jnp.minimum(base + it * CHUNK, E_PAD - CHUNK)
        pltpu.async_copy(src_hbm.at[pl.ds(off, CHUNK)], sidx.at[j], isems[j])
        pltpu.async_copy(dst_hbm.at[pl.ds(off, CHUNK)], didx.at[j], isems[j])

    def idx_wait(j):
        pltpu.make_async_copy(src_hbm.at[pl.ds(0, CHUNK)], sidx.at[j],
                              isems[j]).wait()
        pltpu.make_async_copy(dst_hbm.at[pl.ds(0, CHUNK)], didx.at[j],
                              isems[j]).wait()

    def wave(jbase, gsl):
        gds = []
        for j in range(NBUF):
            idx_wait(jbase + j)
            gds.append(pltpu.async_copy(tbl_hbm.at[sidx.at[jbase + j]],
                                        rows.at[j], gsems[j]))
        return gds

    def body(g, carry):
        i0 = 2 * NBUF * g
        gds = wave(0, 0)
        sds = []
        for j in range(NBUF):
            gds[j].wait()
            sds.append(pltpu.async_copy(rows.at[j], acc.at[didx.at[j]],
                                        ssems[j], add=True))
        gds2 = []
        for j in range(NBUF):
            sds[j].wait()
            idx_wait(NBUF + j)
            gds2.append(pltpu.async_copy(tbl_hbm.at[sidx.at[NBUF + j]],
                                         rows.at[j], gsems[j]))
        sds2 = []
        for j in range(NBUF):
            gds2[j].wait()
            sds2.append(pltpu.async_copy(rows.at[j],
                                         acc.at[didx.at[NBUF + j]],
                                         ssems[j], add=True))
        for j in range(NBUF):
            sds2[j].wait()
        return carry

    def reload(g):
        for j in range(2 * NBUF):
            idx_load(2 * NBUF * g + j, j)

    reload(0)

    def body_reload(g, carry):
        body(g, carry)
        reload(g + 1)
        return carry

    lax.fori_loop(0, T // (2 * NBUF) - 1, body_reload, 0)
    body(T // (2 * NBUF) - 1, 0)


def _deg_body(dst_hbm, zn_hbm, d0_hbm, d1_hbm, didx, ones_v, acc, *sems):
    c = lax.axis_index("c")
    s = lax.axis_index("s")
    wid = c * NS + s
    isems, ssems = sems[:NBUF], sems[NBUF:]
    pltpu.sync_copy(zn_hbm.at[pl.ds(s * (NP // NS), NP // NS)],
                    acc.at[pl.ds(s * (NP // NS), NP // NS)])
    for i in range(DCH // LANES):
        ones_v[pl.ds(i * LANES, LANES)] = jnp.full((LANES,), 1.0, jnp.float32)
    plsc.subcore_barrier()

    T = EPW // DCH
    base = wid * EPW

    def idx_load(it, j):
        off = jnp.minimum(base + it * DCH, E_PAD - DCH)
        pltpu.async_copy(dst_hbm.at[pl.ds(off, DCH)], didx.at[j], isems[j])

    def group(i0, reload):
        sds = []
        for j in range(NBUF):
            pltpu.make_async_copy(dst_hbm.at[pl.ds(0, DCH)], didx.at[j],
                                  isems[j]).wait()
            sds.append(pltpu.async_copy(ones_v, acc.at[didx.at[j]],
                                        ssems[j], add=True))
        for j in range(NBUF):
            sds[j].wait()
        if reload:
            for j in range(NBUF):
                idx_load(i0 + NBUF + j, j)

    for j in range(NBUF):
        idx_load(j, j)

    def body(g, carry):
        group(NBUF * g, True)
        return carry

    lax.fori_loop(0, T // NBUF - 1, body, 0)
    group(T - NBUF, False)
    plsc.subcore_barrier()

    @pl.when(jnp.logical_and(s == 0, c == 0))
    def _():
        pltpu.sync_copy(acc, d0_hbm)

    @pl.when(jnp.logical_and(s == 0, c == 1))
    def _():
        pltpu.sync_copy(acc, d1_hbm)


def _make_deg():
    return pl.kernel(
        _deg_body,
        out_type=(jax.ShapeDtypeStruct((NP,), jnp.float32),
                  jax.ShapeDtypeStruct((NP,), jnp.float32)),
        mesh=plsc.VectorSubcoreMesh(**_MESH),
        scratch_types=[
            pltpu.VMEM((NBUF, DCH), jnp.int32),
            pltpu.VMEM((DCH,), jnp.float32),
            pltpu.VMEM_SHARED((NP,), jnp.float32),
        ] + [pltpu.SemaphoreType.DMA] * (2 * NBUF),
    )


def _mp1_body(src_hbm, dst_hbm, xt_hbm, z_hbm, p0_hbm, p1_hbm,
              sidx, didx, rows, acc, *sems):
    c = lax.axis_index("c")
    s = lax.axis_index("s")
    wid = c * NS + s
    r0 = s * RPT

    @pl.when(c == 0)
    def _():
        pltpu.sync_copy(xt_hbm.at[pl.ds(r0, RPT)], acc.at[pl.ds(r0, RPT)])

    @pl.when(c == 1)
    def _():
        pltpu.sync_copy(z_hbm.at[pl.ds(r0, RPT)], acc.at[pl.ds(r0, RPT)])

    plsc.subcore_barrier()
    _edge_pass(src_hbm, dst_hbm, xt_hbm, acc, sidx, didx, rows,
               sems[:2 * NBUF], sems[2 * NBUF:3 * NBUF], sems[3 * NBUF:],
               wid * EPW, EPW // CHUNK)
    plsc.subcore_barrier()

    @pl.when(c == 0)
    def _():
        pltpu.sync_copy(acc.at[pl.ds(r0, RPT)], p0_hbm.at[pl.ds(r0, RPT)])

    @pl.when(c == 1)
    def _():
        pltpu.sync_copy(acc.at[pl.ds(r0, RPT)], p1_hbm.at[pl.ds(r0, RPT)])


def _make_mp1():
    return pl.kernel(
        _mp1_body,
        out_type=(jax.ShapeDtypeStruct((PR, D_IN), jnp.float32),
                  jax.ShapeDtypeStruct((PR, D_IN), jnp.float32)),
        mesh=plsc.VectorSubcoreMesh(**_MESH),
        scratch_types=[
            pltpu.VMEM((2 * NBUF, CHUNK), jnp.int32),
            pltpu.VMEM((2 * NBUF, CHUNK), jnp.int32),
            pltpu.VMEM((NBUF, CHUNK, D_IN), jnp.float32),
            pltpu.VMEM_SHARED((PR, D_IN), jnp.float32),
        ] + [pltpu.SemaphoreType.DMA] * (4 * NBUF),
    )


def _mp2_body(src_hbm, dst_hbm, m0, m1, m2, m3, v0, v1, v2, v3,
              sidx, didx, rows, acc, *sems):
    c = lax.axis_index("c")
    s = lax.axis_index("s")
    r0 = s * RPT

    def quarter(m_hbm, v_hbm):
        pltpu.sync_copy(m_hbm.at[pl.ds(r0, RPT)], acc.at[pl.ds(r0, RPT)])
        plsc.subcore_barrier()
        _edge_pass(src_hbm, dst_hbm, m_hbm, acc, sidx, didx, rows,
                   sems[:2 * NBUF], sems[2 * NBUF:3 * NBUF], sems[3 * NBUF:],
                   s * EPT, EPT // CHUNK)
        plsc.subcore_barrier()
        pltpu.sync_copy(acc.at[pl.ds(r0, RPT)], v_hbm.at[pl.ds(r0, RPT)])

    @pl.when(c == 0)
    def _():
        quarter(m0, v0)
        quarter(m1, v1)

    @pl.when(c == 1)
    def _():
        quarter(m2, v2)
        quarter(m3, v3)


def _make_mp2():
    vec = jax.ShapeDtypeStruct((PR, D_IN), jnp.float32)
    return pl.kernel(
        _mp2_body,
        out_type=(vec, vec, vec, vec),
        mesh=plsc.VectorSubcoreMesh(**_MESH),
        scratch_types=[
            pltpu.VMEM((2 * NBUF, CHUNK), jnp.int32),
            pltpu.VMEM((2 * NBUF, CHUNK), jnp.int32),
            pltpu.VMEM((NBUF, CHUNK, D_IN), jnp.float32),
            pltpu.VMEM_SHARED((PR, D_IN), jnp.float32),
        ] + [pltpu.SemaphoreType.DMA] * (4 * NBUF),
    )


def _prep_body(d0_ref, d1_ref, x_ref, r_ref, xt_ref):
    d = d0_ref[...] + d1_ref[...] + 1.0
    rr = lax.rsqrt(d)
    r_ref[...] = rr
    xt_ref[0:N, :] = x_ref[...] * rr[:N]
    xt_ref[N:PR, :] = jnp.zeros((PR - N, D_IN), jnp.float32)


def _make_prep():
    return pl.pallas_call(
        _prep_body,
        out_shape=(jax.ShapeDtypeStruct((PR, 1), jnp.float32),
                   jax.ShapeDtypeStruct((PR, D_IN), jnp.float32)),
    )


BC = 2048


def _mm_body(p0_ref, p1_ref, r_ref, W1_ref, b1_ref, W2_ref,
             m0_ref, m1_ref, m2_ref, m3_ref):
    r = r_ref[...]
    u = (p0_ref[...] + p1_ref[...]) * r
    h = jnp.dot(u, W1_ref[...], preferred_element_type=jnp.float32)
    h = jnp.maximum(h + b1_ref[...][None, :], 0.0)
    mt = jnp.dot(h, W2_ref[...], preferred_element_type=jnp.float32) * r
    m0_ref[...] = mt[:, 0:128]
    m1_ref[...] = mt[:, 128:256]
    m2_ref[...] = mt[:, 256:384]
    m3_ref[...] = mt[:, 384:512]


def _make_mm():
    blk = pl.BlockSpec((BC, D_IN), lambda i: (i, 0))
    return pl.pallas_call(
        _mm_body,
        grid=(PR // BC,),
        in_specs=[
            blk, blk,
            pl.BlockSpec((BC, 1), lambda i: (i, 0)),
            pl.BlockSpec((D_IN, D_H), lambda i: (0, 0)),
            pl.BlockSpec((D_H,), lambda i: (0,)),
            pl.BlockSpec((D_H, D_H), lambda i: (0, 0)),
        ],
        out_specs=[blk, blk, blk, blk],
        out_shape=tuple(jax.ShapeDtypeStruct((PR, D_IN), jnp.float32)
                        for _ in range(4)),
    )


def _pool_body(v0_ref, v1_ref, v2_ref, v3_ref, r_ref, b2_ref, bt_ref,
               out_ref, sums, counts):
    i = pl.program_id(0)

    @pl.when(i == 0)
    def _():
        sums[...] = jnp.zeros_like(sums)
        counts[...] = jnp.zeros_like(counts)

    bb = bt_ref[...]
    gi = lax.broadcasted_iota(jnp.int32, (BC, G), 1)
    oh = (bb == gi).astype(jnp.float32)
    ones = jnp.ones((BC, 1), jnp.float32)
    dn = (((0,), (0,)), ((), ()))
    counts[...] += lax.dot_general(oh, ones, dn,
                                   preferred_element_type=jnp.float32)
    r = r_ref[...]
    for q, v_ref in enumerate((v0_ref, v1_ref, v2_ref, v3_ref)):
        h2 = jnp.maximum(v_ref[...] * r + b2_ref[...][None,
                                                      q * 128:(q + 1) * 128],
                         0.0)
        sums[q] += lax.dot_general(oh, h2, dn,
                                   preferred_element_type=jnp.float32)

    @pl.when(i == pl.num_programs(0) - 1)
    def _():
        cts = jnp.maximum(counts[...], 1.0)
        for q in range(4):
            out_ref[:, q * 128:(q + 1) * 128] = sums[q] / cts


def _make_pool():
    blk = pl.BlockSpec((BC, D_IN), lambda i: (i, 0))
    return pl.pallas_call(
        _pool_body,
        grid=(PR // BC,),
        in_specs=[
            blk, blk, blk, blk,
            pl.BlockSpec((BC, 1), lambda i: (i, 0)),
            pl.BlockSpec((D_H,), lambda i: (0,)),
            pl.BlockSpec((BC, 1), lambda i: (i, 0)),
        ],
        out_specs=pl.BlockSpec((G, D_H), lambda i: (0, 0)),
        out_shape=jax.ShapeDtypeStruct((G, D_H), jnp.float32),
        scratch_shapes=[
            pltpu.VMEM((4, G, D_IN), jnp.float32),
            pltpu.VMEM((G, 1), jnp.float32),
        ],
    )


@functools.partial(jax.jit, static_argnums=())
def _run(x, src, dst, batch2d, W1, b1, W2, b2):
    pad = E_PAD - src.shape[0]
    ar = lax.iota(jnp.int32, pad)
    src_p = jnp.concatenate([src, ar % 64])
    dst_p = jnp.concatenate([dst, N + (ar % 64)])
    zn = jnp.zeros((NP,), jnp.float32)
    z128 = jnp.zeros((PR, D_IN), jnp.float32)

    d0, d1 = _make_deg()(dst_p, zn)
    r, xt = _make_prep()(d0.reshape(NP, 1), d1.reshape(NP, 1), x)
    p0, p1 = _make_mp1()(src_p, dst_p, xt, z128)
    m0, m1, m2, m3 = _make_mm()(p0, p1, r, W1, b1, W2)
    v0, v1, v2, v3 = _make_mp2()(src_p, dst_p, m0, m1, m2, m3)
    return _make_pool()(v0, v1, v2, v3, r, b2, batch2d)


def kernel(x, edge_index, batch, W1, b1, W2, b2):
    src = edge_index[0].astype(jnp.int32)
    dst = edge_index[1].astype(jnp.int32)
    bpad = jnp.full((PR - N,), G, jnp.int32)
    batch2d = jnp.concatenate([batch.astype(jnp.int32), bpad]).reshape(PR, 1)
    return _run(x, src, dst, batch2d, W1, b1, W2, b2)
